# baseline (speedup 1.0000x reference)
import jax
import jax.numpy as jnp
from jax import lax
from jax.experimental import pallas as pl
from jax.experimental.pallas import tpu as pltpu
from jax.experimental.pallas import tpu_sc as plsc

N = 10000
E = 320000
D = 128
SIZE2 = 1024

NC = 2
NS = 16
NW = NC * NS
CHUNK = 128
CPT = 80
NCH_PAD = CPT * NW
E_PAD = NCH_PAD * CHUNK
N_P = 10240
TR = N_P // NS
DUMMY = N


def _mesh():
    return plsc.VectorSubcoreMesh(core_axis_name="c", subcore_axis_name="s",
                                  num_cores=NC, num_subcores=NS)


def _deg_body(col0_hbm, col1_hbm, ones_hbm, zeros_hbm, out_hbm,
              d_sp, ci_v, ones_v):
    c = lax.axis_index("c")
    s = lax.axis_index("s")
    w = c * NS + s
    sl = pl.ds(s * TR, TR)

    pltpu.sync_copy(ones_hbm, ones_v)
    for lyr, col_hbm in ((0, col0_hbm), (1, col1_hbm)):
        pltpu.sync_copy(zeros_hbm.at[sl], d_sp.at[sl])
        plsc.subcore_barrier()

        def step(j, carry):
            ch = w * CPT + j
            pltpu.sync_copy(col_hbm.at[pl.ds(ch * CHUNK, CHUNK)], ci_v)
            pltpu.sync_copy(ones_v, d_sp.at[ci_v], add=True)
            return carry

        lax.fori_loop(0, CPT, step, 0)
        plsc.subcore_barrier()
        pltpu.sync_copy(d_sp.at[sl], out_hbm.at[c, lyr, sl])
        plsc.subcore_barrier()


def _degrees(col0, col1, ones, zeros):
    f = pl.kernel(
        _deg_body,
        out_type=jax.ShapeDtypeStruct((NC, 2, N_P, 16), jnp.float32),
        mesh=_mesh(),
        scratch_types=[
            pltpu.VMEM_SHARED((N_P, 16), jnp.float32),
            pltpu.VMEM((CHUNK,), jnp.int32),
            pltpu.VMEM((CHUNK, 16), jnp.float32),
        ],
    )
    return f(col0, col1, ones, zeros)


SCW = 8
NSUPER = CPT // SCW


def _scat_body(hs_hbm, ridx_hbm, cidx_hbm, out_hbm,
               acc_sp, rib, cib, rows0, rows1, sem0, sem1):
    c = lax.axis_index("c")
    s = lax.axis_index("s")
    w = c * NS + s
    sl = pl.ds(s * TR, TR)

    pltpu.sync_copy(hs_hbm.at[sl], acc_sp.at[sl])
    plsc.subcore_barrier()

    def super_body(S, carry):
        base = w * CPT + S * SCW
        pltpu.sync_copy(ridx_hbm.at[pl.ds(base, SCW)], rib)
        pltpu.sync_copy(cidx_hbm.at[pl.ds(base, SCW)], cib)
        handles = [None, None]
        handles[0] = pltpu.async_copy(hs_hbm.at[rib.at[0]], rows0, sem0)
        for jj in range(SCW):
            cur = rows0 if jj % 2 == 0 else rows1
            if jj + 1 < SCW:
                nxt, nsem = (rows1, sem1) if jj % 2 == 0 else (rows0, sem0)
                handles[(jj + 1) % 2] = pltpu.async_copy(
                    hs_hbm.at[rib.at[jj + 1]], nxt, nsem)
            handles[jj % 2].wait()
            pltpu.sync_copy(cur, acc_sp.at[cib.at[jj]], add=True)
        return carry

    lax.fori_loop(0, NSUPER, super_body, 0)
    plsc.subcore_barrier()
    pltpu.sync_copy(acc_sp.at[sl], out_hbm.at[c, sl])


def _scatter(hs, ridx, cidx):
    f = pl.kernel(
        _scat_body,
        out_type=jax.ShapeDtypeStruct((NC, N_P, D), jnp.float32),
        mesh=_mesh(),
        scratch_types=[
            pltpu.VMEM_SHARED((N_P, D), jnp.float32),
            pltpu.VMEM((SCW, CHUNK), jnp.int32),
            pltpu.VMEM((SCW, CHUNK), jnp.int32),
            pltpu.VMEM((CHUNK, D), jnp.float32),
            pltpu.VMEM((CHUNK, D), jnp.float32),
            pltpu.SemaphoreType.DMA,
            pltpu.SemaphoreType.DMA,
        ],
    )
    return f(hs, ridx, cidx)


BN = 1000
GRID = N // BN


def _dinv_from(dc, layer):
    deg = dc[0, layer, :, 0] + dc[1, layer, :, 0] + 1.0
    return lax.rsqrt(deg)[:, None]


def _mm_body(x_ref, w_ref, o_ref):
    o_ref[...] = jnp.dot(x_ref[...], w_ref[...],
                         preferred_element_type=jnp.float32)


def _mm(x, W1):
    return pl.pallas_call(
        _mm_body,
        grid=(GRID,),
        in_specs=[
            pl.BlockSpec((BN, D), lambda i: (i, 0)),
            pl.BlockSpec((D, D), lambda i: (0, 0)),
        ],
        out_specs=pl.BlockSpec((BN, D), lambda i: (i, 0)),
        out_shape=jax.ShapeDtypeStruct((N_P, D), jnp.float32),
    )(x, W1)


def _scale_body(xw_ref, dc_ref, o_ref):
    o_ref[...] = _dinv_from(dc_ref[...], 0) * xw_ref[...]


def _scale(xw, dc):
    return pl.pallas_call(
        _scale_body,
        grid=(GRID,),
        in_specs=[
            pl.BlockSpec((BN, D), lambda i: (i, 0)),
            pl.BlockSpec((NC, 2, BN, 16), lambda i: (0, 0, i, 0)),
        ],
        out_specs=pl.BlockSpec((BN, D), lambda i: (i, 0)),
        out_shape=jax.ShapeDtypeStruct((N_P, D), jnp.float32),
    )(xw, dc)


def _mid_body(ap_ref, hs_ref, dc_ref, w_ref, b_ref, o_ref):
    dc = dc_ref[...]
    a = ap_ref[0] + ap_ref[1] - hs_ref[...]
    h1 = jnp.maximum(_dinv_from(dc, 0) * a + b_ref[...], 0.0)
    h2 = jnp.dot(h1, w_ref[...], preferred_element_type=jnp.float32)
    o_ref[...] = _dinv_from(dc, 1) * h2


def _mid(accp, hs1, dc, W2, b1r):
    return pl.pallas_call(
        _mid_body,
        grid=(GRID,),
        in_specs=[
            pl.BlockSpec((NC, BN, D), lambda i: (0, i, 0)),
            pl.BlockSpec((BN, D), lambda i: (i, 0)),
            pl.BlockSpec((NC, 2, BN, 16), lambda i: (0, 0, i, 0)),
            pl.BlockSpec((D, D), lambda i: (0, 0)),
            pl.BlockSpec((1, D), lambda i: (0, 0)),
        ],
        out_specs=pl.BlockSpec((BN, D), lambda i: (i, 0)),
        out_shape=jax.ShapeDtypeStruct((N_P, D), jnp.float32),
    )(accp, hs1, dc, W2, b1r)


def _dec_body(ap_ref, hs_ref, dc_ref, b_ref, lz_ref, adj_ref):
    a = ap_ref[0] + ap_ref[1] - hs_ref[...]
    z = _dinv_from(dc_ref[...], 1) * a + b_ref[...]
    adj_ref[...] = lax.dot_general(z, z, (((1,), (1,)), ((), ())),
                                   preferred_element_type=jnp.float32)
    m = jnp.max(z, axis=-1, keepdims=True)
    ez = jnp.exp(z - m)
    lz_ref[...] = (z - m) - jnp.log(jnp.sum(ez, axis=-1, keepdims=True))


def _decoder(accp2, hs2, dc, b2r):
    return pl.pallas_call(
        _dec_body,
        grid=(1,),
        in_specs=[
            pl.BlockSpec((NC, SIZE2, D), lambda i: (0, 0, 0)),
            pl.BlockSpec((SIZE2, D), lambda i: (0, 0)),
            pl.BlockSpec((NC, 2, SIZE2, 16), lambda i: (0, 0, 0, 0)),
            pl.BlockSpec((1, D), lambda i: (0, 0)),
        ],
        out_specs=[
            pl.BlockSpec((SIZE2, D), lambda i: (0, 0)),
            pl.BlockSpec((SIZE2, SIZE2), lambda i: (0, 0)),
        ],
        out_shape=[
            jax.ShapeDtypeStruct((SIZE2, D), jnp.float32),
            jax.ShapeDtypeStruct((SIZE2, SIZE2), jnp.float32),
        ],
    )(accp2, hs2, dc, b2r)


def _pad_idx(v, fill):
    pad = jnp.full((E_PAD - E,), fill, jnp.int32)
    return jnp.concatenate([v.astype(jnp.int32), pad]).reshape(NCH_PAD, CHUNK)


def kernel(x, edge_index0, edge_index1, W1, b1, W2, b2, size1_dst, size2_dst):
    ridx0 = _pad_idx(edge_index0[0], 0)
    cidx0 = _pad_idx(edge_index0[1], DUMMY)
    ridx1 = _pad_idx(edge_index1[0], 0)
    cidx1 = _pad_idx(edge_index1[1], DUMMY)
    ones = jnp.ones((CHUNK, 16), jnp.float32)
    zeros = jnp.zeros((N_P, 16), jnp.float32)
    b1r = b1.reshape(1, D)
    b2r = b2.reshape(1, D)

    xw = _mm(x, W1)
    dc = _degrees(cidx0.reshape(-1), cidx1.reshape(-1), ones, zeros)
    hs1 = _scale(xw, dc)
    accp1 = _scatter(hs1, ridx0, cidx0)
    hs2 = _mid(accp1, hs1, dc, W2, b1r)
    accp2 = _scatter(hs2, ridx1, cidx1)
    lz, adj = _decoder(accp2, hs2, dc, b2r)
    return (lz, adj)


# speedup vs baseline: 1.1802x; 1.1802x over previous
import jax
import jax.numpy as jnp
from jax import lax
from jax.experimental import pallas as pl
from jax.experimental.pallas import tpu as pltpu
from jax.experimental.pallas import tpu_sc as plsc

N = 10000
E = 320000
D = 128
SIZE2 = 1024

NC = 2
NS = 16
NW = NC * NS
CHUNK = 128
CPT = 80
NCH_PAD = CPT * NW
E_PAD = NCH_PAD * CHUNK
N_P = 10240
TR = N_P // NS
DUMMY = N


def _mesh():
    return plsc.VectorSubcoreMesh(core_axis_name="c", subcore_axis_name="s",
                                  num_cores=NC, num_subcores=NS)


def _deg_body(col0_hbm, col1_hbm, ones_hbm, zeros_hbm, out_hbm,
              d_sp, ci_v, ones_v):
    c = lax.axis_index("c")
    s = lax.axis_index("s")
    w = c * NS + s
    sl = pl.ds(s * TR, TR)

    pltpu.sync_copy(ones_hbm, ones_v)
    for lyr, col_hbm in ((0, col0_hbm), (1, col1_hbm)):
        pltpu.sync_copy(zeros_hbm.at[sl], d_sp.at[sl])
        plsc.subcore_barrier()

        def step(j, carry):
            ch = w * CPT + j
            pltpu.sync_copy(col_hbm.at[pl.ds(ch * CHUNK, CHUNK)], ci_v)
            pltpu.sync_copy(ones_v, d_sp.at[ci_v], add=True)
            return carry

        lax.fori_loop(0, CPT, step, 0)
        plsc.subcore_barrier()
        pltpu.sync_copy(d_sp.at[sl], out_hbm.at[c, lyr, sl])
        plsc.subcore_barrier()


def _degrees(col0, col1, ones, zeros):
    f = pl.kernel(
        _deg_body,
        out_type=jax.ShapeDtypeStruct((NC, 2, N_P, 16), jnp.float32),
        mesh=_mesh(),
        scratch_types=[
            pltpu.VMEM_SHARED((N_P, 16), jnp.float32),
            pltpu.VMEM((CHUNK,), jnp.int32),
            pltpu.VMEM((CHUNK, 16), jnp.float32),
        ],
    )
    return f(col0, col1, ones, zeros)


SCW = 8
NSUPER = CPT // SCW


def _scat_body(hs_hbm, ridx_hbm, cidx_hbm, out_hbm,
               acc_sp, rib, cib, rows0, rows1, sem0, sem1):
    c = lax.axis_index("c")
    s = lax.axis_index("s")
    w = c * NS + s
    sl = pl.ds(s * TR, TR)

    pltpu.sync_copy(hs_hbm.at[sl], acc_sp.at[sl])
    plsc.subcore_barrier()

    def super_body(S, carry):
        base = w * CPT + S * SCW
        pltpu.sync_copy(ridx_hbm.at[pl.ds(base, SCW)], rib)
        pltpu.sync_copy(cidx_hbm.at[pl.ds(base, SCW)], cib)
        handles = [None, None]
        handles[0] = pltpu.async_copy(hs_hbm.at[rib.at[0]], rows0, sem0)
        for jj in range(SCW):
            cur = rows0 if jj % 2 == 0 else rows1
            if jj + 1 < SCW:
                nxt, nsem = (rows1, sem1) if jj % 2 == 0 else (rows0, sem0)
                handles[(jj + 1) % 2] = pltpu.async_copy(
                    hs_hbm.at[rib.at[jj + 1]], nxt, nsem)
            handles[jj % 2].wait()
            pltpu.sync_copy(cur, acc_sp.at[cib.at[jj]], add=True)
        return carry

    lax.fori_loop(0, NSUPER, super_body, 0)
    plsc.subcore_barrier()
    pltpu.sync_copy(acc_sp.at[sl], out_hbm.at[c, sl])


def _scatter(hs, ridx, cidx):
    f = pl.kernel(
        _scat_body,
        out_type=jax.ShapeDtypeStruct((NC, N_P, D), jnp.float32),
        mesh=_mesh(),
        scratch_types=[
            pltpu.VMEM_SHARED((N_P, D), jnp.float32),
            pltpu.VMEM((SCW, CHUNK), jnp.int32),
            pltpu.VMEM((SCW, CHUNK), jnp.int32),
            pltpu.VMEM((CHUNK, D), jnp.float32),
            pltpu.VMEM((CHUNK, D), jnp.float32),
            pltpu.SemaphoreType.DMA,
            pltpu.SemaphoreType.DMA,
        ],
    )
    return f(hs, ridx, cidx)


BN = 1000
GRID = N // BN


def _dinv_from(dc, layer):
    deg = dc[0, layer, :, 0] + dc[1, layer, :, 0] + 1.0
    return lax.rsqrt(deg)[:, None]


def _mm_body(x_ref, w_ref, o_ref):
    o_ref[...] = jnp.dot(x_ref[...], w_ref[...],
                         preferred_element_type=jnp.float32)


def _mm(x, W1):
    return pl.pallas_call(
        _mm_body,
        grid=(GRID,),
        in_specs=[
            pl.BlockSpec((BN, D), lambda i: (i, 0)),
            pl.BlockSpec((D, D), lambda i: (0, 0)),
        ],
        out_specs=pl.BlockSpec((BN, D), lambda i: (i, 0)),
        out_shape=jax.ShapeDtypeStruct((N_P, D), jnp.float32),
    )(x, W1)


def _scale_body(xw_ref, dc_ref, o_ref):
    o_ref[...] = _dinv_from(dc_ref[...], 0) * xw_ref[...]


def _scale(xw, dc):
    return pl.pallas_call(
        _scale_body,
        grid=(GRID,),
        in_specs=[
            pl.BlockSpec((BN, D), lambda i: (i, 0)),
            pl.BlockSpec((NC, 2, BN, 16), lambda i: (0, 0, i, 0)),
        ],
        out_specs=pl.BlockSpec((BN, D), lambda i: (i, 0)),
        out_shape=jax.ShapeDtypeStruct((N_P, D), jnp.float32),
    )(xw, dc)


def _mid_body(ap_ref, hs_ref, dc_ref, w_ref, b_ref, o_ref):
    dc = dc_ref[...]
    a = ap_ref[0] + ap_ref[1] - hs_ref[...]
    h1 = jnp.maximum(_dinv_from(dc, 0) * a + b_ref[...], 0.0)
    h2 = jnp.dot(h1, w_ref[...], preferred_element_type=jnp.float32)
    o_ref[...] = _dinv_from(dc, 1) * h2


def _mid(accp, hs1, dc, W2, b1r):
    return pl.pallas_call(
        _mid_body,
        grid=(GRID,),
        in_specs=[
            pl.BlockSpec((NC, BN, D), lambda i: (0, i, 0)),
            pl.BlockSpec((BN, D), lambda i: (i, 0)),
            pl.BlockSpec((NC, 2, BN, 16), lambda i: (0, 0, i, 0)),
            pl.BlockSpec((D, D), lambda i: (0, 0)),
            pl.BlockSpec((1, D), lambda i: (0, 0)),
        ],
        out_specs=pl.BlockSpec((BN, D), lambda i: (i, 0)),
        out_shape=jax.ShapeDtypeStruct((N_P, D), jnp.float32),
    )(accp, hs1, dc, W2, b1r)


def _dec_body(ap_ref, hs_ref, dc_ref, b_ref, lz_ref, adj_ref):
    a = ap_ref[0] + ap_ref[1] - hs_ref[...]
    z = _dinv_from(dc_ref[...], 1) * a + b_ref[...]
    adj_ref[...] = lax.dot_general(z, z, (((1,), (1,)), ((), ())),
                                   preferred_element_type=jnp.float32)
    m = jnp.max(z, axis=-1, keepdims=True)
    ez = jnp.exp(z - m)
    lz_ref[...] = (z - m) - jnp.log(jnp.sum(ez, axis=-1, keepdims=True))


def _decoder(accp2, hs2, dc, b2r):
    return pl.pallas_call(
        _dec_body,
        grid=(1,),
        in_specs=[
            pl.BlockSpec((NC, SIZE2, D), lambda i: (0, 0, 0)),
            pl.BlockSpec((SIZE2, D), lambda i: (0, 0)),
            pl.BlockSpec((NC, 2, SIZE2, 16), lambda i: (0, 0, 0, 0)),
            pl.BlockSpec((1, D), lambda i: (0, 0)),
        ],
        out_specs=[
            pl.BlockSpec((SIZE2, D), lambda i: (0, 0)),
            pl.BlockSpec((SIZE2, SIZE2), lambda i: (0, 0)),
        ],
        out_shape=[
            jax.ShapeDtypeStruct((SIZE2, D), jnp.float32),
            jax.ShapeDtypeStruct((SIZE2, SIZE2), jnp.float32),
        ],
    )(accp2, hs2, dc, b2r)


def _pad_idx(v, pad):
    return jnp.concatenate([v.astype(jnp.int32), pad]).reshape(NCH_PAD, CHUNK)


def kernel(x, edge_index0, edge_index1, W1, b1, W2, b2, size1_dst, size2_dst):
    pad_r = jnp.zeros((E_PAD - E,), jnp.int32)
    pad_c = N + (jnp.arange(E_PAD - E, dtype=jnp.int32) % (N_P - N))
    ridx0 = _pad_idx(edge_index0[0], pad_r)
    cidx0 = _pad_idx(edge_index0[1], pad_c)
    ridx1 = _pad_idx(edge_index1[0], pad_r)
    cidx1 = _pad_idx(edge_index1[1], pad_c)
    ones = jnp.ones((CHUNK, 16), jnp.float32)
    zeros = jnp.zeros((N_P, 16), jnp.float32)
    b1r = b1.reshape(1, D)
    b2r = b2.reshape(1, D)

    xw = _mm(x, W1)
    dc = _degrees(cidx0.reshape(-1), cidx1.reshape(-1), ones, zeros)
    hs1 = _scale(xw, dc)
    accp1 = _scatter(hs1, ridx0, cidx0)
    hs2 = _mid(accp1, hs1, dc, W2, b1r)
    accp2 = _scatter(hs2, ridx1, cidx1)
    lz, adj = _decoder(accp2, hs2, dc, b2r)
    return (lz, adj)


# speedup vs baseline: 1.1890x; 1.0074x over previous
import jax
import jax.numpy as jnp
from jax import lax
from jax.experimental import pallas as pl
from jax.experimental.pallas import tpu as pltpu
from jax.experimental.pallas import tpu_sc as plsc

N = 10000
E = 320000
D = 128
SIZE2 = 1024

NC = 2
NS = 16
NW = NC * NS
CHUNK = 128
CPT = 80
NCH_PAD = CPT * NW
E_PAD = NCH_PAD * CHUNK
N_P = 10240
TR = N_P // NS
DUMMY = N


def _mesh():
    return plsc.VectorSubcoreMesh(core_axis_name="c", subcore_axis_name="s",
                                  num_cores=NC, num_subcores=NS)


def _deg_body(col0_hbm, col1_hbm, ones_hbm, zeros_hbm, out_hbm,
              d_sp, cib, ones_v):
    c = lax.axis_index("c")
    s = lax.axis_index("s")
    w = c * NS + s
    sl = pl.ds(s * TR, TR)

    pltpu.sync_copy(ones_hbm, ones_v)
    for lyr, col_hbm in ((0, col0_hbm), (1, col1_hbm)):
        pltpu.sync_copy(zeros_hbm.at[sl], d_sp.at[sl])
        plsc.subcore_barrier()

        def super_body(S, carry):
            base = w * CPT + S * SCW
            pltpu.sync_copy(col_hbm.at[pl.ds(base, SCW)], cib)
            for jj in range(SCW):
                pltpu.sync_copy(ones_v, d_sp.at[cib.at[jj]], add=True)
            return carry

        lax.fori_loop(0, NSUPER, super_body, 0)
        plsc.subcore_barrier()
        pltpu.sync_copy(d_sp.at[sl], out_hbm.at[c, lyr, sl])
        plsc.subcore_barrier()


def _degrees(col0, col1, ones, zeros):
    f = pl.kernel(
        _deg_body,
        out_type=jax.ShapeDtypeStruct((NC, 2, N_P, D), jnp.float32),
        mesh=_mesh(),
        scratch_types=[
            pltpu.VMEM_SHARED((N_P, D), jnp.float32),
            pltpu.VMEM((SCW, CHUNK), jnp.int32),
            pltpu.VMEM((CHUNK, D), jnp.float32),
        ],
    )
    return f(col0, col1, ones, zeros)


SCW = 8
NSUPER = CPT // SCW


def _scat_body(hs_hbm, ridx_hbm, cidx_hbm, out_hbm,
               acc_sp, rib, cib, rows0, rows1, sem0, sem1):
    c = lax.axis_index("c")
    s = lax.axis_index("s")
    w = c * NS + s
    sl = pl.ds(s * TR, TR)

    pltpu.sync_copy(hs_hbm.at[sl], acc_sp.at[sl])
    plsc.subcore_barrier()

    def super_body(S, carry):
        base = w * CPT + S * SCW
        pltpu.sync_copy(ridx_hbm.at[pl.ds(base, SCW)], rib)
        pltpu.sync_copy(cidx_hbm.at[pl.ds(base, SCW)], cib)
        handles = [None, None]
        handles[0] = pltpu.async_copy(hs_hbm.at[rib.at[0]], rows0, sem0)
        for jj in range(SCW):
            cur = rows0 if jj % 2 == 0 else rows1
            if jj + 1 < SCW:
                nxt, nsem = (rows1, sem1) if jj % 2 == 0 else (rows0, sem0)
                handles[(jj + 1) % 2] = pltpu.async_copy(
                    hs_hbm.at[rib.at[jj + 1]], nxt, nsem)
            handles[jj % 2].wait()
            pltpu.sync_copy(cur, acc_sp.at[cib.at[jj]], add=True)
        return carry

    lax.fori_loop(0, NSUPER, super_body, 0)
    plsc.subcore_barrier()
    pltpu.sync_copy(acc_sp.at[sl], out_hbm.at[c, sl])


def _scatter(hs, ridx, cidx):
    f = pl.kernel(
        _scat_body,
        out_type=jax.ShapeDtypeStruct((NC, N_P, D), jnp.float32),
        mesh=_mesh(),
        scratch_types=[
            pltpu.VMEM_SHARED((N_P, D), jnp.float32),
            pltpu.VMEM((SCW, CHUNK), jnp.int32),
            pltpu.VMEM((SCW, CHUNK), jnp.int32),
            pltpu.VMEM((CHUNK, D), jnp.float32),
            pltpu.VMEM((CHUNK, D), jnp.float32),
            pltpu.SemaphoreType.DMA,
            pltpu.SemaphoreType.DMA,
        ],
    )
    return f(hs, ridx, cidx)


BN = 1000
GRID = N // BN


def _dinv_from(dc, layer):
    deg = dc[0, layer] + dc[1, layer] + 1.0
    return lax.rsqrt(deg)


def _mm_body(x_ref, w_ref, o_ref):
    o_ref[...] = jnp.dot(x_ref[...], w_ref[...],
                         preferred_element_type=jnp.float32)


def _mm(x, W1):
    return pl.pallas_call(
        _mm_body,
        grid=(GRID,),
        in_specs=[
            pl.BlockSpec((BN, D), lambda i: (i, 0)),
            pl.BlockSpec((D, D), lambda i: (0, 0)),
        ],
        out_specs=pl.BlockSpec((BN, D), lambda i: (i, 0)),
        out_shape=jax.ShapeDtypeStruct((N_P, D), jnp.float32),
    )(x, W1)


def _scale_body(xw_ref, dc_ref, o_ref):
    o_ref[...] = _dinv_from(dc_ref[...], 0) * xw_ref[...]


def _scale(xw, dc):
    return pl.pallas_call(
        _scale_body,
        grid=(GRID,),
        in_specs=[
            pl.BlockSpec((BN, D), lambda i: (i, 0)),
            pl.BlockSpec((NC, 2, BN, D), lambda i: (0, 0, i, 0)),
        ],
        out_specs=pl.BlockSpec((BN, D), lambda i: (i, 0)),
        out_shape=jax.ShapeDtypeStruct((N_P, D), jnp.float32),
    )(xw, dc)


def _mid_body(ap_ref, hs_ref, dc_ref, w_ref, b_ref, o_ref):
    dc = dc_ref[...]
    a = ap_ref[0] + ap_ref[1] - hs_ref[...]
    h1 = jnp.maximum(_dinv_from(dc, 0) * a + b_ref[...], 0.0)
    h2 = jnp.dot(h1, w_ref[...], preferred_element_type=jnp.float32)
    o_ref[...] = _dinv_from(dc, 1) * h2


def _mid(accp, hs1, dc, W2, b1r):
    return pl.pallas_call(
        _mid_body,
        grid=(GRID,),
        in_specs=[
            pl.BlockSpec((NC, BN, D), lambda i: (0, i, 0)),
            pl.BlockSpec((BN, D), lambda i: (i, 0)),
            pl.BlockSpec((NC, 2, BN, D), lambda i: (0, 0, i, 0)),
            pl.BlockSpec((D, D), lambda i: (0, 0)),
            pl.BlockSpec((1, D), lambda i: (0, 0)),
        ],
        out_specs=pl.BlockSpec((BN, D), lambda i: (i, 0)),
        out_shape=jax.ShapeDtypeStruct((N_P, D), jnp.float32),
    )(accp, hs1, dc, W2, b1r)


def _dec_body(ap_ref, hs_ref, dc_ref, b_ref, lz_ref, adj_ref):
    a = ap_ref[0] + ap_ref[1] - hs_ref[...]
    z = _dinv_from(dc_ref[...], 1) * a + b_ref[...]
    adj_ref[...] = lax.dot_general(z, z, (((1,), (1,)), ((), ())),
                                   preferred_element_type=jnp.float32)
    m = jnp.max(z, axis=-1, keepdims=True)
    ez = jnp.exp(z - m)
    lz_ref[...] = (z - m) - jnp.log(jnp.sum(ez, axis=-1, keepdims=True))


def _decoder(accp2, hs2, dc, b2r):
    return pl.pallas_call(
        _dec_body,
        grid=(1,),
        in_specs=[
            pl.BlockSpec((NC, SIZE2, D), lambda i: (0, 0, 0)),
            pl.BlockSpec((SIZE2, D), lambda i: (0, 0)),
            pl.BlockSpec((NC, 2, SIZE2, D), lambda i: (0, 0, 0, 0)),
            pl.BlockSpec((1, D), lambda i: (0, 0)),
        ],
        out_specs=[
            pl.BlockSpec((SIZE2, D), lambda i: (0, 0)),
            pl.BlockSpec((SIZE2, SIZE2), lambda i: (0, 0)),
        ],
        out_shape=[
            jax.ShapeDtypeStruct((SIZE2, D), jnp.float32),
            jax.ShapeDtypeStruct((SIZE2, SIZE2), jnp.float32),
        ],
    )(accp2, hs2, dc, b2r)


def _pad_idx(v, pad):
    return jnp.concatenate([v.astype(jnp.int32), pad]).reshape(NCH_PAD, CHUNK)


def kernel(x, edge_index0, edge_index1, W1, b1, W2, b2, size1_dst, size2_dst):
    pad_r = jnp.zeros((E_PAD - E,), jnp.int32)
    pad_c = N + (jnp.arange(E_PAD - E, dtype=jnp.int32) % (N_P - N))
    ridx0 = _pad_idx(edge_index0[0], pad_r)
    cidx0 = _pad_idx(edge_index0[1], pad_c)
    ridx1 = _pad_idx(edge_index1[0], pad_r)
    cidx1 = _pad_idx(edge_index1[1], pad_c)
    ones = jnp.ones((CHUNK, D), jnp.float32)
    zeros = jnp.zeros((N_P, D), jnp.float32)
    b1r = b1.reshape(1, D)
    b2r = b2.reshape(1, D)

    xw = _mm(x, W1)
    dc = _degrees(cidx0, cidx1, ones, zeros)
    hs1 = _scale(xw, dc)
    accp1 = _scatter(hs1, ridx0, cidx0)
    hs2 = _mid(accp1, hs1, dc, W2, b1r)
    accp2 = _scatter(hs2, ridx1, cidx1)
    lz, adj = _decoder(accp2, hs2, dc, b2r)
    return (lz, adj)


# speedup vs baseline: 2.5606x; 2.1536x over previous
import jax
import jax.numpy as jnp
from jax import lax
from jax.experimental import pallas as pl
from jax.experimental.pallas import tpu as pltpu
from jax.experimental.pallas import tpu_sc as plsc

N = 10000
E = 320000
D = 128
SIZE2 = 1024

NC = 2
NS = 16
NW = NC * NS
CHUNK = 128
CPT = 80
NCH_PAD = CPT * NW
E_PAD = NCH_PAD * CHUNK
N_P = 10240
TR = N_P // NS
DUMMY = N


def _mesh():
    return plsc.VectorSubcoreMesh(core_axis_name="c", subcore_axis_name="s",
                                  num_cores=NC, num_subcores=NS)


def _deg_body(col0_hbm, col1_hbm, ones_hbm, zeros_hbm, out_hbm,
              d_sp, cib, ones_v):
    c = lax.axis_index("c")
    s = lax.axis_index("s")
    w = c * NS + s
    sl = pl.ds(s * TR, TR)

    pltpu.sync_copy(ones_hbm, ones_v)
    for lyr, col_hbm in ((0, col0_hbm), (1, col1_hbm)):
        pltpu.sync_copy(zeros_hbm.at[sl], d_sp.at[sl])
        plsc.subcore_barrier()

        def super_body(S, carry):
            base = w * CPT + S * SCW
            pltpu.sync_copy(col_hbm.at[pl.ds(base, SCW)], cib)
            for jj in range(SCW):
                pltpu.sync_copy(ones_v, d_sp.at[cib.at[jj]], add=True)
            return carry

        lax.fori_loop(0, NSUPER, super_body, 0)
        plsc.subcore_barrier()
        pltpu.sync_copy(d_sp.at[sl], out_hbm.at[c, lyr, sl])
        plsc.subcore_barrier()


def _degrees(col0, col1, ones, zeros):
    f = pl.kernel(
        _deg_body,
        out_type=jax.ShapeDtypeStruct((NC, 2, N_P, D), jnp.float32),
        mesh=_mesh(),
        scratch_types=[
            pltpu.VMEM_SHARED((N_P, D), jnp.float32),
            pltpu.VMEM((SCW, CHUNK), jnp.int32),
            pltpu.VMEM((CHUNK, D), jnp.float32),
        ],
    )
    return f(col0, col1, ones, zeros)


SCW = 8
NSUPER = CPT // SCW


def _scat_body(hs_hbm, ridx_hbm, cidx_hbm, out_hbm,
               acc_sp, rib, cib, rows0, rows1, sem0, sem1):
    c = lax.axis_index("c")
    s = lax.axis_index("s")
    w = c * NS + s
    sl = pl.ds(s * TR, TR)

    pltpu.sync_copy(hs_hbm.at[sl], acc_sp.at[sl])
    plsc.subcore_barrier()

    def super_body(S, carry):
        base = w * CPT + S * SCW
        pltpu.sync_copy(ridx_hbm.at[pl.ds(base, SCW)], rib)
        pltpu.sync_copy(cidx_hbm.at[pl.ds(base, SCW)], cib)
        handles = [None, None]
        handles[0] = pltpu.async_copy(hs_hbm.at[rib.at[0]], rows0, sem0)
        for jj in range(SCW):
            cur = rows0 if jj % 2 == 0 else rows1
            if jj + 1 < SCW:
                nxt, nsem = (rows1, sem1) if jj % 2 == 0 else (rows0, sem0)
                handles[(jj + 1) % 2] = pltpu.async_copy(
                    hs_hbm.at[rib.at[jj + 1]], nxt, nsem)
            handles[jj % 2].wait()
            pltpu.sync_copy(cur, acc_sp.at[cib.at[jj]], add=True)
        return carry

    lax.fori_loop(0, NSUPER, super_body, 0)
    plsc.subcore_barrier()
    pltpu.sync_copy(acc_sp.at[sl], out_hbm.at[c, sl])


def _scatter(hs, ridx, cidx):
    f = pl.kernel(
        _scat_body,
        out_type=jax.ShapeDtypeStruct((NC, N_P, D), jnp.float32),
        mesh=_mesh(),
        scratch_types=[
            pltpu.VMEM_SHARED((N_P, D), jnp.float32),
            pltpu.VMEM((SCW, CHUNK), jnp.int32),
            pltpu.VMEM((SCW, CHUNK), jnp.int32),
            pltpu.VMEM((CHUNK, D), jnp.float32),
            pltpu.VMEM((CHUNK, D), jnp.float32),
            pltpu.SemaphoreType.DMA,
            pltpu.SemaphoreType.DMA,
        ],
    )
    return f(hs, ridx, cidx)


BN = 1000
GRID = N // BN


def _dinv_from(dc, layer):
    deg = dc[0, layer] + dc[1, layer] + 1.0
    return lax.rsqrt(deg)


def _mm_body(x_ref, w_ref, o_ref):
    o_ref[...] = jnp.dot(x_ref[...], w_ref[...],
                         preferred_element_type=jnp.float32)


def _mm(x, W1):
    return pl.pallas_call(
        _mm_body,
        grid=(GRID,),
        in_specs=[
            pl.BlockSpec((BN, D), lambda i: (i, 0)),
            pl.BlockSpec((D, D), lambda i: (0, 0)),
        ],
        out_specs=pl.BlockSpec((BN, D), lambda i: (i, 0)),
        out_shape=jax.ShapeDtypeStruct((N_P, D), jnp.float32),
    )(x, W1)


def _scale_body(xw_ref, dc_ref, o_ref):
    o_ref[...] = _dinv_from(dc_ref[...], 0) * xw_ref[...]


def _scale(xw, dc):
    return pl.pallas_call(
        _scale_body,
        grid=(GRID,),
        in_specs=[
            pl.BlockSpec((BN, D), lambda i: (i, 0)),
            pl.BlockSpec((NC, 2, BN, D), lambda i: (0, 0, i, 0)),
        ],
        out_specs=pl.BlockSpec((BN, D), lambda i: (i, 0)),
        out_shape=jax.ShapeDtypeStruct((N_P, D), jnp.float32),
    )(xw, dc)


def _mid_body(ap_ref, hs_ref, dc_ref, w_ref, b_ref, o_ref):
    dc = dc_ref[...]
    a = ap_ref[0] + ap_ref[1] - hs_ref[...]
    h1 = jnp.maximum(_dinv_from(dc, 0) * a + b_ref[...], 0.0)
    h2 = jnp.dot(h1, w_ref[...], preferred_element_type=jnp.float32)
    o_ref[...] = _dinv_from(dc, 1) * h2


def _mid(accp, hs1, dc, W2, b1r):
    return pl.pallas_call(
        _mid_body,
        grid=(GRID,),
        in_specs=[
            pl.BlockSpec((NC, BN, D), lambda i: (0, i, 0)),
            pl.BlockSpec((BN, D), lambda i: (i, 0)),
            pl.BlockSpec((NC, 2, BN, D), lambda i: (0, 0, i, 0)),
            pl.BlockSpec((D, D), lambda i: (0, 0)),
            pl.BlockSpec((1, D), lambda i: (0, 0)),
        ],
        out_specs=pl.BlockSpec((BN, D), lambda i: (i, 0)),
        out_shape=jax.ShapeDtypeStruct((N_P, D), jnp.float32),
    )(accp, hs1, dc, W2, b1r)


def _dec_body(ap_ref, hs_ref, dc_ref, b_ref, lz_ref, adj_ref):
    a = ap_ref[0] + ap_ref[1] - hs_ref[...]
    z = _dinv_from(dc_ref[...], 1) * a + b_ref[...]
    adj_ref[...] = lax.dot_general(z, z, (((1,), (1,)), ((), ())),
                                   preferred_element_type=jnp.float32)
    m = jnp.max(z, axis=-1, keepdims=True)
    ez = jnp.exp(z - m)
    lz_ref[...] = (z - m) - jnp.log(jnp.sum(ez, axis=-1, keepdims=True))


def _decoder(accp2, hs2, dc, b2r):
    return pl.pallas_call(
        _dec_body,
        grid=(1,),
        in_specs=[
            pl.BlockSpec((NC, SIZE2, D), lambda i: (0, 0, 0)),
            pl.BlockSpec((SIZE2, D), lambda i: (0, 0)),
            pl.BlockSpec((NC, 2, SIZE2, D), lambda i: (0, 0, 0, 0)),
            pl.BlockSpec((1, D), lambda i: (0, 0)),
        ],
        out_specs=[
            pl.BlockSpec((SIZE2, D), lambda i: (0, 0)),
            pl.BlockSpec((SIZE2, SIZE2), lambda i: (0, 0)),
        ],
        out_shape=[
            jax.ShapeDtypeStruct((SIZE2, D), jnp.float32),
            jax.ShapeDtypeStruct((SIZE2, SIZE2), jnp.float32),
        ],
    )(accp2, hs2, dc, b2r)


def _pad_idx(v, pad):
    return jnp.concatenate([v.astype(jnp.int32), pad]).reshape(NCH_PAD, CHUNK)


def kernel(x, edge_index0, edge_index1, W1, b1, W2, b2, size1_dst, size2_dst):
    pad_i = jnp.arange(E_PAD - E, dtype=jnp.int32)
    pad_r = pad_i % N
    pad_c = N + pad_i % (N_P - N)
    ridx0 = _pad_idx(edge_index0[0], pad_r)
    cidx0 = _pad_idx(edge_index0[1], pad_c)
    ridx1 = _pad_idx(edge_index1[0], pad_r)
    cidx1 = _pad_idx(edge_index1[1], pad_c)
    ones = jnp.ones((CHUNK, D), jnp.float32)
    zeros = jnp.zeros((N_P, D), jnp.float32)
    b1r = b1.reshape(1, D)
    b2r = b2.reshape(1, D)

    xw = _mm(x, W1)
    dc = _degrees(cidx0, cidx1, ones, zeros)
    hs1 = _scale(xw, dc)
    accp1 = _scatter(hs1, ridx0, cidx0)
    hs2 = _mid(accp1, hs1, dc, W2, b1r)
    accp2 = _scatter(hs2, ridx1, cidx1)
    lz, adj = _decoder(accp2, hs2, dc, b2r)
    return (lz, adj)
